# baseline (speedup 1.0000x reference)
import functools

import jax
import jax.numpy as jnp
from jax import lax
from jax.experimental import pallas as pl
from jax.experimental.pallas import tpu as pltpu
from jax.experimental.pallas import tpu_sc as plsc

D_MODEL = 768
SEQ_LEN = 8192
LANES = 16
KREG = D_MODEL // LANES
KB = 8
N_TOK = 32768
NW = 32
TPW = N_TOK // NW
CH = 16
NBUF = 8
NCH = TPW // CH


def _sc_embed(seg_2d, table_flat):
    mesh = plsc.VectorSubcoreMesh(core_axis_name="c", subcore_axis_name="s")

    @functools.partial(
        pl.kernel,
        mesh=mesh,
        out_type=jax.ShapeDtypeStruct((N_TOK, D_MODEL), jnp.float32),
        scratch_types=[
            pltpu.VMEM((TPW,), jnp.int32),
            pltpu.VMEM((2, D_MODEL), jnp.float32),
            pltpu.VMEM((D_MODEL,), jnp.float32),
        ]
        + [pltpu.VMEM((CH, D_MODEL), jnp.float32) for _ in range(NBUF)]
        + [pltpu.SemaphoreType.DMA for _ in range(NBUF)],
    )
    def k(seg_hbm, tab_hbm, out_hbm, seg_v, tab_v, dif_v, *obs_sems):
        obs = obs_sems[:NBUF]
        sems = obs_sems[NBUF:]
        wid = lax.axis_index("s") * 2 + lax.axis_index("c")
        base = wid * TPW
        wpb = SEQ_LEN // TPW
        pltpu.sync_copy(tab_hbm, tab_v)
        pltpu.sync_copy(
            seg_hbm.at[wid // wpb, pl.ds((wid % wpb) * TPW, TPW)], seg_v
        )
        for kk in range(KREG):
            dif_v[pl.ds(kk * LANES, LANES)] = (
                tab_v[1, pl.ds(kk * LANES, LANES)]
                - tab_v[0, pl.ds(kk * LANES, LANES)]
            )

        def splat(fgrp, j):
            jv = lax.broadcast(j, (LANES,))
            return lax.gather(
                fgrp,
                jv[:, None],
                dimension_numbers=lax.GatherDimensionNumbers(
                    offset_dims=(),
                    collapsed_slice_dims=(0,),
                    start_index_map=(0,),
                ),
                slice_sizes=(1,),
                mode=lax.GatherScatterMode.PROMISE_IN_BOUNDS,
            )

        def fill(c, ob):
            def kb_body(kb, carry):
                cb = kb * KB * LANES
                t0s = [
                    tab_v[0, pl.ds(cb + i * LANES, LANES)] for i in range(KB)
                ]
                dfs = [dif_v[pl.ds(cb + i * LANES, LANES)] for i in range(KB)]

                def grp(g, carry2):
                    fgrp = seg_v[pl.ds(c * CH + g * LANES, LANES)].astype(
                        jnp.float32
                    )

                    def tok(j, carry3):
                        fj = splat(fgrp, j)
                        row = g * LANES + j
                        for i in range(KB):
                            ob[row, pl.ds(cb + i * LANES, LANES)] = (
                                t0s[i] + fj * dfs[i]
                            )
                        return carry3

                    lax.fori_loop(0, LANES, tok, 0)
                    return carry2

                lax.fori_loop(0, CH // LANES, grp, 0)
                return carry

            lax.fori_loop(0, KREG // KB, kb_body, 0)

        def flush(c, ob, sem):
            return pltpu.async_copy(
                ob, out_hbm.at[pl.ds(base + c * CH, CH)], sem
            )

        def drain(ob, sem):
            pltpu.make_async_copy(ob, out_hbm.at[pl.ds(base, CH)], sem).wait()

        def body(q, carry):
            c0 = q * NBUF
            for b in range(NBUF):

                @pl.when(q > 0)
                def _(b=b):
                    drain(obs[b], sems[b])

                fill(c0 + b, obs[b])
                flush(c0 + b, obs[b], sems[b])
            return carry

        lax.fori_loop(0, NCH // NBUF, body, 0)
        for b in range(NBUF):
            drain(obs[b], sems[b])

    return k(seg_2d, table_flat)


def kernel(segment_ids, table):
    b, s = segment_ids.shape
    out = _sc_embed(segment_ids.astype(jnp.int32), table)
    return out.reshape(b, s, D_MODEL)


# speedup vs baseline: 1.0176x; 1.0176x over previous
import functools

import jax
import jax.numpy as jnp
from jax import lax
from jax.experimental import pallas as pl
from jax.experimental.pallas import tpu as pltpu
from jax.experimental.pallas import tpu_sc as plsc

D_MODEL = 768
SEQ_LEN = 8192
LANES = 16
KREG = D_MODEL // LANES
KB = 8
N_TOK = 32768
NW = 32
TPW = N_TOK // NW
CH = 32
NBUF = 4
NCH = TPW // CH


def _sc_embed(seg_2d, table_flat):
    mesh = plsc.VectorSubcoreMesh(core_axis_name="c", subcore_axis_name="s")

    @functools.partial(
        pl.kernel,
        mesh=mesh,
        out_type=jax.ShapeDtypeStruct((N_TOK, D_MODEL), jnp.float32),
        scratch_types=[
            pltpu.VMEM((TPW,), jnp.int32),
            pltpu.VMEM((2, D_MODEL), jnp.float32),
            pltpu.VMEM((D_MODEL,), jnp.float32),
        ]
        + [pltpu.VMEM((CH, D_MODEL), jnp.float32) for _ in range(NBUF)]
        + [pltpu.SemaphoreType.DMA for _ in range(NBUF)],
    )
    def k(seg_hbm, tab_hbm, out_hbm, seg_v, tab_v, dif_v, *obs_sems):
        obs = obs_sems[:NBUF]
        sems = obs_sems[NBUF:]
        wid = lax.axis_index("s") * 2 + lax.axis_index("c")
        base = wid * TPW
        wpb = SEQ_LEN // TPW
        pltpu.sync_copy(tab_hbm, tab_v)
        pltpu.sync_copy(
            seg_hbm.at[wid // wpb, pl.ds((wid % wpb) * TPW, TPW)], seg_v
        )
        for kk in range(KREG):
            dif_v[pl.ds(kk * LANES, LANES)] = (
                tab_v[1, pl.ds(kk * LANES, LANES)]
                - tab_v[0, pl.ds(kk * LANES, LANES)]
            )

        def splat(fgrp, j):
            jv = lax.broadcast(j, (LANES,))
            return lax.gather(
                fgrp,
                jv[:, None],
                dimension_numbers=lax.GatherDimensionNumbers(
                    offset_dims=(),
                    collapsed_slice_dims=(0,),
                    start_index_map=(0,),
                ),
                slice_sizes=(1,),
                mode=lax.GatherScatterMode.PROMISE_IN_BOUNDS,
            )

        def fill(c, ob):
            def kb_body(kb, carry):
                cb = kb * KB * LANES
                t0s = [
                    tab_v[0, pl.ds(cb + i * LANES, LANES)] for i in range(KB)
                ]
                dfs = [dif_v[pl.ds(cb + i * LANES, LANES)] for i in range(KB)]

                def grp(g, carry2):
                    fgrp = seg_v[pl.ds(c * CH + g * LANES, LANES)].astype(
                        jnp.float32
                    )

                    def tok(j, carry3):
                        fj = splat(fgrp, j)
                        row = g * LANES + j
                        for i in range(KB):
                            ob[row, pl.ds(cb + i * LANES, LANES)] = (
                                t0s[i] + fj * dfs[i]
                            )
                        return carry3

                    lax.fori_loop(0, LANES, tok, 0)
                    return carry2

                lax.fori_loop(0, CH // LANES, grp, 0)
                return carry

            lax.fori_loop(0, KREG // KB, kb_body, 0)

        def flush(c, ob, sem):
            return pltpu.async_copy(
                ob, out_hbm.at[pl.ds(base + c * CH, CH)], sem
            )

        def drain(ob, sem):
            pltpu.make_async_copy(ob, out_hbm.at[pl.ds(base, CH)], sem).wait()

        def body(q, carry):
            c0 = q * NBUF
            for b in range(NBUF):

                @pl.when(q > 0)
                def _(b=b):
                    drain(obs[b], sems[b])

                fill(c0 + b, obs[b])
                flush(c0 + b, obs[b], sems[b])
            return carry

        lax.fori_loop(0, NCH // NBUF, body, 0)
        for b in range(NBUF):
            drain(obs[b], sems[b])

    return k(seg_2d, table_flat)


def kernel(segment_ids, table):
    b, s = segment_ids.shape
    out = _sc_embed(segment_ids.astype(jnp.int32), table)
    return out.reshape(b, s, D_MODEL)
